# baseline (speedup 1.0000x reference)
import jax
import jax.numpy as jnp
from jax import lax
from jax.experimental import pallas as pl
from jax.experimental.pallas import tpu as pltpu

DIMS = 128
M = 512
TB = 8


def _colstats_kernel(q_ref, k_ref, colmax_ref, colsum_ref, qt_ref,
                     m_scr, s_scr):
    i = pl.program_id(0)
    nt = pl.num_programs(0)
    q3 = q_ref[...]
    K = k_ref[...]
    score = lax.dot_general(q3, K, (((0,), (1,)), ((), ())),
                            preferred_element_type=jnp.float32)
    tmax = jnp.max(jnp.max(score, axis=0), axis=0, keepdims=True)

    eye = (lax.broadcasted_iota(jnp.int32, (DIMS, DIMS), 0)
           == lax.broadcasted_iota(jnp.int32, (DIMS, DIMS), 1)
           ).astype(jnp.float32)
    qt_ref[...] = lax.dot_general(q3, eye, (((0,), (0,)), ((), ())),
                                  precision=lax.Precision.HIGHEST,
                                  preferred_element_type=jnp.float32)

    @pl.when(i == 0)
    def _():
        m_scr[...] = jnp.full_like(m_scr, -jnp.inf)
        s_scr[...] = jnp.zeros_like(s_scr)

    m_old = m_scr[...]
    m_new = jnp.maximum(m_old, tmax)
    scale = jnp.exp(m_old - m_new)
    s_new = s_scr[...] * scale + jnp.sum(
        jnp.sum(jnp.exp(score - m_new[None]), axis=0), axis=0, keepdims=True)
    m_scr[...] = m_new
    s_scr[...] = s_new

    @pl.when(i == nt - 1)
    def _():
        colmax_ref[...] = m_new
        colsum_ref[...] = s_new


def _main_kernel(q_ref, qt_ref, k_ref, colmax_ref, colsum_ref,
                 uq_ref, um_ref, smq_ref, smm_ref, spread_ref, gloss_ref,
                 acc_scr):
    i = pl.program_id(0)
    nt = pl.num_programs(0)
    q3 = q_ref[...]
    K = k_ref[...]
    C = q3.shape[2]
    R = TB * C
    qb = qt_ref[...].reshape(R, DIMS)
    score = lax.dot_general(q3, K, (((0,), (1,)), ((), ())),
                            preferred_element_type=jnp.float32)
    score = score.reshape(R, M)

    rowmax = jnp.max(score, axis=1, keepdims=True)
    e = jnp.exp(score - rowmax)
    rowsum = jnp.sum(e, axis=1, keepdims=True)
    smm = e * (1.0 / rowsum)
    smm3 = smm.reshape(TB, C, M)
    smm_ref[...] = smm3

    colmax = colmax_ref[...]
    colsum = colsum_ref[...]
    P2 = (e * jnp.exp(rowmax)) * (jnp.exp(-colmax) / colsum)
    smq_ref[...] = P2.reshape(TB, C, M)

    iota = lax.broadcasted_iota(jnp.int32, (R, M), 1)
    BIG = jnp.int32(2 ** 30)
    g = jnp.min(jnp.where(score == rowmax, iota, BIG), axis=1, keepdims=True)
    mask1 = iota == g
    m1f = mask1.astype(jnp.float32)
    score2 = jnp.where(mask1, -jnp.inf, score)
    row2max = jnp.max(score2, axis=1, keepdims=True)
    g2 = jnp.min(jnp.where(score2 == row2max, iota, BIG), axis=1, keepdims=True)
    m2f = (iota == g2).astype(jnp.float32)

    k1 = lax.dot_general(m1f, K, (((1,), (0,)), ((), ())),
                         preferred_element_type=jnp.float32)
    d1 = qb - k1
    gloss_ref[...] = (d1 * d1).reshape(TB, C, DIMS)

    ksum = jnp.sum(K, axis=1, keepdims=True)
    kss = jnp.sum(K * K, axis=1, keepdims=True)
    c1 = kss - jnp.float32(2e-6) * ksum
    qe = qb + jnp.float32(1e-6)
    sqq = jnp.sum(qe * qe, axis=1, keepdims=True)
    gp = lax.dot_general(m1f, c1, (((1,), (0,)), ((), ())),
                         precision=lax.Precision.HIGHEST,
                         preferred_element_type=jnp.float32)
    gn = lax.dot_general(m2f, c1, (((1,), (0,)), ((), ())),
                         precision=lax.Precision.HIGHEST,
                         preferred_element_type=jnp.float32)
    dp = jnp.sqrt(jnp.maximum(sqq - 2.0 * rowmax + gp, 0.0))
    dn = jnp.sqrt(jnp.maximum(sqq - 2.0 * row2max + gn, 0.0))
    spread_ref[...] = jnp.maximum(dp - dn + 1.0, 0.0).reshape(TB, C)

    cmT = lax.dot_general(K, smm3, (((0,), (2,)), ((), ())),
                          preferred_element_type=jnp.float32)
    uq_ref[0:DIMS, :, :] = q3
    uq_ref[DIMS:2 * DIMS, :, :] = cmT

    w_oh = m1f * P2
    contrib = lax.dot_general(w_oh, qb, (((0,), (0,)), ((), ())),
                              preferred_element_type=jnp.float32)

    @pl.when(i == 0)
    def _():
        acc_scr[...] = jnp.zeros_like(acc_scr)

    acc_scr[...] += contrib

    @pl.when(i == nt - 1)
    def _():
        colsum_col = jnp.transpose(colsum)
        upd = acc_scr[...] * colsum_col + K
        nrm = jnp.sqrt(jnp.sum(upd * upd, axis=1, keepdims=True))
        um_ref[...] = upd / jnp.maximum(nrm, 1e-12)


def kernel(query, keys):
    dims, t, n = query.shape
    m = keys.shape[0]
    N = t * n
    assert dims == DIMS and m == M and t % TB == 0

    C1 = 1024
    G1t, G1n = t // TB, n // C1
    colmax, colsum, qt = pl.pallas_call(
        _colstats_kernel,
        grid=(G1t * G1n,),
        in_specs=[
            pl.BlockSpec((dims, TB, C1), lambda i: (0, i // G1n, i % G1n)),
            pl.BlockSpec((m, dims), lambda i: (0, 0)),
        ],
        out_specs=[
            pl.BlockSpec((1, m), lambda i: (0, 0)),
            pl.BlockSpec((1, m), lambda i: (0, 0)),
            pl.BlockSpec((TB, C1, dims), lambda i: (i // G1n, i % G1n, 0)),
        ],
        out_shape=[
            jax.ShapeDtypeStruct((1, m), jnp.float32),
            jax.ShapeDtypeStruct((1, m), jnp.float32),
            jax.ShapeDtypeStruct((t, n, dims), jnp.float32),
        ],
        scratch_shapes=[
            pltpu.VMEM((1, m), jnp.float32),
            pltpu.VMEM((1, m), jnp.float32),
        ],
        compiler_params=pltpu.CompilerParams(
            dimension_semantics=("arbitrary",)),
    )(query, keys)

    C = 256
    Gt, Gn = t // TB, n // C
    uq, um, smq, smm, spread, gloss = pl.pallas_call(
        _main_kernel,
        grid=(Gt * Gn,),
        in_specs=[
            pl.BlockSpec((dims, TB, C), lambda i: (0, i // Gn, i % Gn)),
            pl.BlockSpec((TB, C, dims), lambda i: (i // Gn, i % Gn, 0)),
            pl.BlockSpec((m, dims), lambda i: (0, 0)),
            pl.BlockSpec((1, m), lambda i: (0, 0)),
            pl.BlockSpec((1, m), lambda i: (0, 0)),
        ],
        out_specs=[
            pl.BlockSpec((2 * dims, TB, C), lambda i: (0, i // Gn, i % Gn)),
            pl.BlockSpec((m, dims), lambda i: (0, 0)),
            pl.BlockSpec((TB, C, m), lambda i: (i // Gn, i % Gn, 0)),
            pl.BlockSpec((TB, C, m), lambda i: (i // Gn, i % Gn, 0)),
            pl.BlockSpec((TB, C), lambda i: (i // Gn, i % Gn)),
            pl.BlockSpec((TB, C, dims), lambda i: (i // Gn, i % Gn, 0)),
        ],
        out_shape=[
            jax.ShapeDtypeStruct((2 * dims, t, n), jnp.float32),
            jax.ShapeDtypeStruct((m, dims), jnp.float32),
            jax.ShapeDtypeStruct((t, n, m), jnp.float32),
            jax.ShapeDtypeStruct((t, n, m), jnp.float32),
            jax.ShapeDtypeStruct((t, n), jnp.float32),
            jax.ShapeDtypeStruct((t, n, dims), jnp.float32),
        ],
        scratch_shapes=[
            pltpu.VMEM((m, dims), jnp.float32),
        ],
        compiler_params=pltpu.CompilerParams(
            dimension_semantics=("arbitrary",)),
    )(query, qt, keys, colmax, colsum)

    return (uq, um, smq.reshape(N, m), smm.reshape(N, m),
            spread.reshape(N), gloss.reshape(N, dims))


# speedup vs baseline: 1.5641x; 1.5641x over previous
import jax
import jax.numpy as jnp
from jax import lax
from jax.experimental import pallas as pl
from jax.experimental.pallas import tpu as pltpu

DIMS = 128
M = 512
TB = 8


def _colstats_kernel(q_ref, k_ref, colmax_ref, colsum_ref, qt_ref,
                     m_scr, s_scr):
    i = pl.program_id(0)
    nt = pl.num_programs(0)
    q3 = q_ref[...]
    K = k_ref[...]
    score = lax.dot_general(q3, K, (((0,), (1,)), ((), ())),
                            preferred_element_type=jnp.float32)
    tmax = jnp.max(jnp.max(score, axis=0), axis=0, keepdims=True)

    eye = (lax.broadcasted_iota(jnp.int32, (DIMS, DIMS), 0)
           == lax.broadcasted_iota(jnp.int32, (DIMS, DIMS), 1)
           ).astype(jnp.float32)
    qt_ref[...] = lax.dot_general(q3, eye, (((0,), (0,)), ((), ())),
                                  precision=lax.Precision.HIGHEST,
                                  preferred_element_type=jnp.float32)

    @pl.when(i == 0)
    def _():
        m_scr[...] = jnp.full_like(m_scr, -jnp.inf)
        s_scr[...] = jnp.zeros_like(s_scr)

    m_old = m_scr[...]
    m_new = jnp.maximum(m_old, tmax)
    scale = jnp.exp(m_old - m_new)
    s_new = s_scr[...] * scale + jnp.sum(
        jnp.sum(jnp.exp(score - m_new[None]), axis=0), axis=0, keepdims=True)
    m_scr[...] = m_new
    s_scr[...] = s_new

    @pl.when(i == nt - 1)
    def _():
        colmax_ref[...] = m_new
        colsum_ref[...] = s_new


def _main_kernel(q_ref, qt_ref, k_ref, colmax_ref, colsum_ref,
                 uq_ref, um_ref, smq_ref, smm_ref, spread_ref, gloss_ref,
                 acc_scr):
    i = pl.program_id(0)
    nt = pl.num_programs(0)
    q3 = q_ref[...]
    K = k_ref[...]
    C = q3.shape[2]
    R = TB * C
    qb = qt_ref[...].reshape(R, DIMS)
    score = lax.dot_general(q3, K, (((0,), (1,)), ((), ())),
                            preferred_element_type=jnp.float32)
    score = score.reshape(R, M)

    rowmax = jnp.max(score, axis=1, keepdims=True)
    e = jnp.exp(score - rowmax)
    rowsum = jnp.sum(e, axis=1, keepdims=True)
    smm = e * (1.0 / rowsum)
    smm3 = smm.reshape(TB, C, M)
    smm_ref[...] = smm3

    colmax = colmax_ref[...]
    colsum = colsum_ref[...]
    P2 = (e * jnp.exp(rowmax)) * (jnp.exp(-colmax) / colsum)
    smq_ref[...] = P2.reshape(TB, C, M)

    iota = lax.broadcasted_iota(jnp.int32, (R, M), 1)
    BIG = jnp.int32(2 ** 30)
    g = jnp.min(jnp.where(score == rowmax, iota, BIG), axis=1, keepdims=True)
    mask1 = iota == g
    m1f = mask1.astype(jnp.float32)
    score2 = jnp.where(mask1, -jnp.inf, score)
    row2max = jnp.max(score2, axis=1, keepdims=True)
    g2 = jnp.min(jnp.where(score2 == row2max, iota, BIG), axis=1, keepdims=True)
    m2f = (iota == g2).astype(jnp.float32)

    k1 = lax.dot_general(m1f, K, (((1,), (0,)), ((), ())),
                         preferred_element_type=jnp.float32)
    d1 = qb - k1
    gloss_ref[...] = (d1 * d1).reshape(TB, C, DIMS)

    k2 = lax.dot_general(m2f, K, (((1,), (0,)), ((), ())),
                         preferred_element_type=jnp.float32)
    dp = jnp.sqrt(jnp.sum((d1 + 1e-6) ** 2, axis=1, keepdims=True))
    dn = jnp.sqrt(jnp.sum((qb - k2 + 1e-6) ** 2, axis=1, keepdims=True))
    spread_ref[...] = jnp.maximum(dp - dn + 1.0, 0.0).reshape(TB, C)

    cmT = lax.dot_general(K, smm3, (((0,), (2,)), ((), ())),
                          preferred_element_type=jnp.float32)
    uq_ref[0:DIMS, :, :] = q3
    uq_ref[DIMS:2 * DIMS, :, :] = cmT

    w_oh = m1f * P2
    contrib = lax.dot_general(w_oh, qb, (((0,), (0,)), ((), ())),
                              preferred_element_type=jnp.float32)

    @pl.when(i == 0)
    def _():
        acc_scr[...] = jnp.zeros_like(acc_scr)

    acc_scr[...] += contrib

    @pl.when(i == nt - 1)
    def _():
        colsum_col = jnp.transpose(colsum)
        upd = acc_scr[...] * colsum_col + K
        nrm = jnp.sqrt(jnp.sum(upd * upd, axis=1, keepdims=True))
        um_ref[...] = upd / jnp.maximum(nrm, 1e-12)


def kernel(query, keys):
    dims, t, n = query.shape
    m = keys.shape[0]
    N = t * n
    assert dims == DIMS and m == M and t % TB == 0

    C1 = 1024
    G1t, G1n = t // TB, n // C1
    colmax, colsum, qt = pl.pallas_call(
        _colstats_kernel,
        grid=(G1t * G1n,),
        in_specs=[
            pl.BlockSpec((dims, TB, C1), lambda i: (0, i // G1n, i % G1n)),
            pl.BlockSpec((m, dims), lambda i: (0, 0)),
        ],
        out_specs=[
            pl.BlockSpec((1, m), lambda i: (0, 0)),
            pl.BlockSpec((1, m), lambda i: (0, 0)),
            pl.BlockSpec((TB, C1, dims), lambda i: (i // G1n, i % G1n, 0)),
        ],
        out_shape=[
            jax.ShapeDtypeStruct((1, m), jnp.float32),
            jax.ShapeDtypeStruct((1, m), jnp.float32),
            jax.ShapeDtypeStruct((t, n, dims), jnp.float32),
        ],
        scratch_shapes=[
            pltpu.VMEM((1, m), jnp.float32),
            pltpu.VMEM((1, m), jnp.float32),
        ],
        compiler_params=pltpu.CompilerParams(
            dimension_semantics=("arbitrary",)),
    )(query, keys)

    C = 256
    Gt, Gn = t // TB, n // C
    uq, um, smq, smm, spread, gloss = pl.pallas_call(
        _main_kernel,
        grid=(Gt * Gn,),
        in_specs=[
            pl.BlockSpec((dims, TB, C), lambda i: (0, i // Gn, i % Gn)),
            pl.BlockSpec((TB, C, dims), lambda i: (i // Gn, i % Gn, 0)),
            pl.BlockSpec((m, dims), lambda i: (0, 0)),
            pl.BlockSpec((1, m), lambda i: (0, 0)),
            pl.BlockSpec((1, m), lambda i: (0, 0)),
        ],
        out_specs=[
            pl.BlockSpec((2 * dims, TB, C), lambda i: (0, i // Gn, i % Gn)),
            pl.BlockSpec((m, dims), lambda i: (0, 0)),
            pl.BlockSpec((TB, C, m), lambda i: (i // Gn, i % Gn, 0)),
            pl.BlockSpec((TB, C, m), lambda i: (i // Gn, i % Gn, 0)),
            pl.BlockSpec((TB, C), lambda i: (i // Gn, i % Gn)),
            pl.BlockSpec((TB, C, dims), lambda i: (i // Gn, i % Gn, 0)),
        ],
        out_shape=[
            jax.ShapeDtypeStruct((2 * dims, t, n), jnp.float32),
            jax.ShapeDtypeStruct((m, dims), jnp.float32),
            jax.ShapeDtypeStruct((t, n, m), jnp.float32),
            jax.ShapeDtypeStruct((t, n, m), jnp.float32),
            jax.ShapeDtypeStruct((t, n), jnp.float32),
            jax.ShapeDtypeStruct((t, n, dims), jnp.float32),
        ],
        scratch_shapes=[
            pltpu.VMEM((m, dims), jnp.float32),
        ],
        compiler_params=pltpu.CompilerParams(
            dimension_semantics=("arbitrary",)),
    )(query, qt, keys, colmax, colsum)

    return (uq, um, smq.reshape(N, m), smm.reshape(N, m),
            spread.reshape(N), gloss.reshape(N, dims))


# speedup vs baseline: 1.6627x; 1.0631x over previous
import jax
import jax.numpy as jnp
from jax import lax
from jax.experimental import pallas as pl
from jax.experimental.pallas import tpu as pltpu

DIMS = 128
M = 512
TB = 8


def _colstats_kernel(q_ref, k_ref, colmax_ref, colsum_ref, m_scr, s_scr):
    i = pl.program_id(0)
    nt = pl.num_programs(0)
    q3 = q_ref[...]
    K = k_ref[...]
    score = lax.dot_general(q3, K, (((0,), (1,)), ((), ())),
                            preferred_element_type=jnp.float32)
    tmax = jnp.max(jnp.max(score, axis=0), axis=0, keepdims=True)

    @pl.when(i == 0)
    def _():
        m_scr[...] = jnp.full_like(m_scr, -jnp.inf)
        s_scr[...] = jnp.zeros_like(s_scr)

    m_scr[...] = jnp.maximum(m_scr[...], tmax)
    s_scr[...] += jnp.sum(jnp.sum(jnp.exp(score), axis=0), axis=0,
                          keepdims=True)

    @pl.when(i == nt - 1)
    def _():
        colmax_ref[...] = m_scr[...]
        colsum_ref[...] = s_scr[...] * jnp.exp(-m_scr[...])


def _main_kernel(q_ref, k_ref, colmax_ref, colsum_ref,
                 uq_ref, um_ref, smq_ref, smm_ref, spread_ref, gloss_ref,
                 acc_scr):
    i = pl.program_id(0)
    nt = pl.num_programs(0)
    q3 = q_ref[...]
    K = k_ref[...]
    C = q3.shape[2]
    R = TB * C
    eye = (lax.broadcasted_iota(jnp.int32, (DIMS, DIMS), 0)
           == lax.broadcasted_iota(jnp.int32, (DIMS, DIMS), 1)
           ).astype(jnp.float32)
    qb = lax.dot_general(q3, eye, (((0,), (0,)), ((), ())),
                         precision=lax.Precision.HIGHEST,
                         preferred_element_type=jnp.float32).reshape(R, DIMS)
    score = lax.dot_general(q3, K, (((0,), (1,)), ((), ())),
                            preferred_element_type=jnp.float32)
    score = score.reshape(R, M)

    rowmax = jnp.max(score, axis=1, keepdims=True)
    e = jnp.exp(score - rowmax)
    rowsum = jnp.sum(e, axis=1, keepdims=True)
    smm = e * (1.0 / rowsum)
    smm3 = smm.reshape(TB, C, M)
    smm_ref[...] = smm3

    colmax = colmax_ref[...]
    colsum = colsum_ref[...]
    P2 = (e * jnp.exp(rowmax)) * (jnp.exp(-colmax) / colsum)
    smq_ref[...] = P2.reshape(TB, C, M)

    iota = lax.broadcasted_iota(jnp.int32, (R, M), 1)
    BIG = jnp.int32(2 ** 30)
    g = jnp.min(jnp.where(score == rowmax, iota, BIG), axis=1, keepdims=True)
    mask1 = iota == g
    m1f = mask1.astype(jnp.float32)
    score2 = jnp.where(mask1, -jnp.inf, score)
    row2max = jnp.max(score2, axis=1, keepdims=True)
    g2 = jnp.min(jnp.where(score2 == row2max, iota, BIG), axis=1, keepdims=True)
    m2f = (iota == g2).astype(jnp.float32)

    k1 = lax.dot_general(m1f, K, (((1,), (0,)), ((), ())),
                         preferred_element_type=jnp.float32)
    d1 = qb - k1
    gloss_ref[...] = (d1 * d1).reshape(TB, C, DIMS)

    k2 = lax.dot_general(m2f, K, (((1,), (0,)), ((), ())),
                         preferred_element_type=jnp.float32)
    dp = jnp.sqrt(jnp.sum((d1 + 1e-6) ** 2, axis=1, keepdims=True))
    dn = jnp.sqrt(jnp.sum((qb - k2 + 1e-6) ** 2, axis=1, keepdims=True))
    spread_ref[...] = jnp.maximum(dp - dn + 1.0, 0.0).reshape(TB, C)

    cmT = lax.dot_general(K, smm3, (((0,), (2,)), ((), ())),
                          preferred_element_type=jnp.float32)
    uq_ref[0:DIMS, :, :] = q3
    uq_ref[DIMS:2 * DIMS, :, :] = cmT

    w_oh = m1f * P2
    contrib = lax.dot_general(w_oh, qb, (((0,), (0,)), ((), ())),
                              preferred_element_type=jnp.float32)

    @pl.when(i == 0)
    def _():
        acc_scr[...] = jnp.zeros_like(acc_scr)

    acc_scr[...] += contrib

    @pl.when(i == nt - 1)
    def _():
        colsum_col = jnp.transpose(colsum)
        upd = acc_scr[...] * colsum_col + K
        nrm = jnp.sqrt(jnp.sum(upd * upd, axis=1, keepdims=True))
        um_ref[...] = upd / jnp.maximum(nrm, 1e-12)


def kernel(query, keys):
    dims, t, n = query.shape
    m = keys.shape[0]
    N = t * n
    assert dims == DIMS and m == M and t % TB == 0

    C1 = 1024
    G1t, G1n = t // TB, n // C1
    colmax, colsum = pl.pallas_call(
        _colstats_kernel,
        grid=(G1t * G1n,),
        in_specs=[
            pl.BlockSpec((dims, TB, C1), lambda i: (0, i // G1n, i % G1n)),
            pl.BlockSpec((m, dims), lambda i: (0, 0)),
        ],
        out_specs=[
            pl.BlockSpec((1, m), lambda i: (0, 0)),
            pl.BlockSpec((1, m), lambda i: (0, 0)),
        ],
        out_shape=[
            jax.ShapeDtypeStruct((1, m), jnp.float32),
            jax.ShapeDtypeStruct((1, m), jnp.float32),
        ],
        scratch_shapes=[
            pltpu.VMEM((1, m), jnp.float32),
            pltpu.VMEM((1, m), jnp.float32),
        ],
        compiler_params=pltpu.CompilerParams(
            dimension_semantics=("arbitrary",)),
    )(query, keys)

    C = 256
    Gt, Gn = t // TB, n // C
    uq, um, smq, smm, spread, gloss = pl.pallas_call(
        _main_kernel,
        grid=(Gt * Gn,),
        in_specs=[
            pl.BlockSpec((dims, TB, C), lambda i: (0, i // Gn, i % Gn)),
            pl.BlockSpec((m, dims), lambda i: (0, 0)),
            pl.BlockSpec((1, m), lambda i: (0, 0)),
            pl.BlockSpec((1, m), lambda i: (0, 0)),
        ],
        out_specs=[
            pl.BlockSpec((2 * dims, TB, C), lambda i: (0, i // Gn, i % Gn)),
            pl.BlockSpec((m, dims), lambda i: (0, 0)),
            pl.BlockSpec((TB, C, m), lambda i: (i // Gn, i % Gn, 0)),
            pl.BlockSpec((TB, C, m), lambda i: (i // Gn, i % Gn, 0)),
            pl.BlockSpec((TB, C), lambda i: (i // Gn, i % Gn)),
            pl.BlockSpec((TB, C, dims), lambda i: (i // Gn, i % Gn, 0)),
        ],
        out_shape=[
            jax.ShapeDtypeStruct((2 * dims, t, n), jnp.float32),
            jax.ShapeDtypeStruct((m, dims), jnp.float32),
            jax.ShapeDtypeStruct((t, n, m), jnp.float32),
            jax.ShapeDtypeStruct((t, n, m), jnp.float32),
            jax.ShapeDtypeStruct((t, n), jnp.float32),
            jax.ShapeDtypeStruct((t, n, dims), jnp.float32),
        ],
        scratch_shapes=[
            pltpu.VMEM((m, dims), jnp.float32),
        ],
        compiler_params=pltpu.CompilerParams(
            dimension_semantics=("arbitrary",)),
    )(query, keys, colmax, colsum)

    return (uq, um, smq.reshape(N, m), smm.reshape(N, m),
            spread.reshape(N), gloss.reshape(N, dims))
